# baseline (speedup 1.0000x reference)
import functools

import jax
import jax.numpy as jnp
from jax import lax
from jax.experimental import pallas as pl
from jax.experimental.pallas import tpu as pltpu
from jax.experimental.pallas import tpu_sc as plsc

H = 224
W = 224
C = 96
MB = 8
NPIX = MB * H * W

NC = 2
NS = 16
NW = NC * NS
ROWS_PER_W = H // NW
HALF = W // 2

_STEP = 2.0 / 223.0


def _bf16r(x):
    u = lax.bitcast_convert_type(x, jnp.int32)
    rnd = lax.bitwise_and(lax.shift_right_logical(u, jnp.int32(16)),
                          jnp.int32(1))
    u = u + jnp.int32(0x7FFF) + rnd
    u = lax.bitwise_and(u, jnp.int32(-65536))
    return lax.bitcast_convert_type(u, jnp.float32)


def _affine_kernel(im_hbm, th_hbm, out_hbm, th_v, idx_v, wa_v, wb_v, wc_v,
                   wd_v, ga_v, gb_v, gc_v, gd_v, out_v, sem):
    wid = lax.axis_index("s") * NC + lax.axis_index("c")
    pltpu.sync_copy(th_hbm, th_v)

    lanes = lax.iota(jnp.int32, 16)

    def do_image(b, _):
        tsel = [_bf16r(
            plsc.load_gather(th_v, [jnp.full((16,), b * 6 + k, jnp.int32)]))
                for k in range(6)]
        t0, t1, t2, t3, t4, t5 = tsel
        base_b = b * (H * W)

        def do_row(j, _):
            r = wid * ROWS_PER_W + j
            ygv = _bf16r(jnp.float32(-1.0) + jnp.full((16,), r, jnp.int32)
                         .astype(jnp.float32) * jnp.float32(_STEP))

            for h in range(2):
                for m in range(HALF // 16):
                    cols = lanes + (h * HALF + m * 16)
                    xgv = _bf16r(jnp.float32(-1.0)
                                 + cols.astype(jnp.float32)
                                 * jnp.float32(_STEP))
                    Xn = (t0 * xgv + t1 * ygv) + t2
                    Yn = (t3 * xgv + t4 * ygv) + t5
                    X = (Xn + 1.0) / 2.0 * jnp.float32(W)
                    Y = (Yn + 1.0) / 2.0 * jnp.float32(H)
                    fx = X.astype(jnp.int32)
                    fx = jnp.where(fx.astype(jnp.float32) > X, fx - 1, fx)
                    fy = Y.astype(jnp.int32)
                    fy = jnp.where(fy.astype(jnp.float32) > Y, fy - 1, fy)
                    x0 = jnp.clip(fx, 0, W - 1)
                    x1 = jnp.clip(fx + 1, 0, W - 1)
                    y0 = jnp.clip(fy, 0, H - 1)
                    y1 = jnp.clip(fy + 1, 0, H - 1)
                    x0f = x0.astype(jnp.float32)
                    x1f = x1.astype(jnp.float32)
                    y0f = y0.astype(jnp.float32)
                    y1f = y1.astype(jnp.float32)
                    sl = pl.ds(m * 16, 16)
                    wa_v[sl] = (x1f - X) * (y1f - Y)
                    wb_v[sl] = (x1f - X) * (Y - y0f)
                    wc_v[sl] = (X - x0f) * (y1f - Y)
                    wd_v[sl] = (X - x0f) * (Y - y0f)
                    ra = base_b + y0 * W
                    rb = base_b + y1 * W
                    idx_v[0, sl] = ra + x0
                    idx_v[1, sl] = rb + x0
                    idx_v[2, sl] = ra + x1
                    idx_v[3, sl] = rb + x1

                cps = [pltpu.async_copy(im_hbm.at[idx_v.at[g]], dst, sem)
                       for g, dst in enumerate((ga_v, gb_v, gc_v, gd_v))]
                for cp in cps:
                    cp.wait()

                def blend(i, _):
                    iv = jnp.full((16,), i, jnp.int32)
                    wav = plsc.load_gather(wa_v, [iv])
                    wbv = plsc.load_gather(wb_v, [iv])
                    wcv = plsc.load_gather(wc_v, [iv])
                    wdv = plsc.load_gather(wd_v, [iv])
                    for n in range(C // 16):
                        csl = pl.ds(n * 16, 16)
                        va = ga_v[i, csl]
                        vb = gb_v[i, csl]
                        vc = gc_v[i, csl]
                        vd = gd_v[i, csl]
                        out_v[pl.ds(i * C + n * 16, 16)] = (
                            va * wav + vb * wbv + vc * wcv + vd * wdv)
                    return 0

                lax.fori_loop(0, HALF, blend, 0)

                pix0 = base_b + r * W + h * HALF
                off = pl.multiple_of(pix0 * C, 8)
                pltpu.sync_copy(out_v, out_hbm.at[pl.ds(off, HALF * C)])
            return 0

        lax.fori_loop(0, ROWS_PER_W, do_row, 0)
        return 0

    lax.fori_loop(0, MB, do_image, 0)


@jax.jit
def _run(im2, th_flat):
    mesh = plsc.VectorSubcoreMesh(core_axis_name="c", subcore_axis_name="s")
    f = functools.partial(
        pl.kernel,
        mesh=mesh,
        compiler_params=pltpu.CompilerParams(
            needs_layout_passes=False, use_tc_tiling_on_sc=False),
        out_type=jax.ShapeDtypeStruct((NPIX * C,), jnp.float32),
        scratch_types=[
            pltpu.VMEM((MB * 6,), jnp.float32),
            pltpu.VMEM((4, HALF), jnp.int32),
            pltpu.VMEM((HALF,), jnp.float32),
            pltpu.VMEM((HALF,), jnp.float32),
            pltpu.VMEM((HALF,), jnp.float32),
            pltpu.VMEM((HALF,), jnp.float32),
            pltpu.VMEM((HALF, C), jnp.float32),
            pltpu.VMEM((HALF, C), jnp.float32),
            pltpu.VMEM((HALF, C), jnp.float32),
            pltpu.VMEM((HALF, C), jnp.float32),
            pltpu.VMEM((HALF * C,), jnp.float32),
            pltpu.SemaphoreType.DMA,
        ],
    )(_affine_kernel)
    return f(im2, th_flat)


def kernel(im, mb_size, thetas):
    im2 = im.reshape(NPIX, C)
    th_flat = thetas.reshape(MB * 6)
    flat = _run(im2, th_flat)
    return flat.reshape(MB, H, W, C)


# speedup vs baseline: 4.2645x; 4.2645x over previous
import functools

import jax
import jax.numpy as jnp
from jax import lax
from jax.experimental import pallas as pl
from jax.experimental.pallas import tpu as pltpu
from jax.experimental.pallas import tpu_sc as plsc

H = 224
W = 224
C = 96
MB = 8
NPIX = MB * H * W

NC = 2
NS = 16
NW = NC * NS
ROWS_PER_W = H // NW
HALF = W // 2
NCHUNK = HALF // 16

_STEP = 2.0 / 223.0


def _bf16r(x):
    u = lax.bitcast_convert_type(x, jnp.int32)
    rnd = lax.bitwise_and(lax.shift_right_logical(u, jnp.int32(16)),
                          jnp.int32(1))
    u = u + jnp.int32(0x7FFF) + rnd
    u = lax.bitwise_and(u, jnp.int32(-65536))
    return lax.bitcast_convert_type(u, jnp.float32)


def _affine_kernel(im_hbm, th_hbm, out_hbm, th_v, idx_v, wa_v, wb_v, wc_v,
                   wd_v, mk_v, ga_v, gb_v, gc_v, gd_v, out_v, sem):
    wid = lax.axis_index("s") * NC + lax.axis_index("c")
    pltpu.sync_copy(th_hbm, th_v)

    lanes = lax.iota(jnp.int32, 16)

    def do_image(b, _):
        tsel = [_bf16r(
            plsc.load_gather(th_v, [jnp.full((16,), b * 6 + k, jnp.int32)]))
                for k in range(6)]
        t0, t1, t2, t3, t4, t5 = tsel
        base_b = b * (H * W)

        def do_row(j, _):
            r = wid * ROWS_PER_W + j
            ygv = _bf16r(jnp.float32(-1.0) + jnp.full((16,), r, jnp.int32)
                         .astype(jnp.float32) * jnp.float32(_STEP))

            for h in range(2):
                for m in range(NCHUNK):
                    cols = lanes + (h * HALF + m * 16)
                    xgv = _bf16r(jnp.float32(-1.0)
                                 + cols.astype(jnp.float32)
                                 * jnp.float32(_STEP))
                    Xn = (t0 * xgv + t1 * ygv) + t2
                    Yn = (t3 * xgv + t4 * ygv) + t5
                    X = (Xn + 1.0) / 2.0 * jnp.float32(W)
                    Y = (Yn + 1.0) / 2.0 * jnp.float32(H)
                    fx = X.astype(jnp.int32)
                    fx = jnp.where(fx.astype(jnp.float32) > X, fx - 1, fx)
                    fy = Y.astype(jnp.int32)
                    fy = jnp.where(fy.astype(jnp.float32) > Y, fy - 1, fy)
                    inr = ((fx >= 0) & (fx <= W - 2)
                           & (fy >= 0) & (fy <= H - 2))
                    x0 = jnp.clip(fx, 0, W - 1)
                    x1 = jnp.clip(fx + 1, 0, W - 1)
                    y0 = jnp.clip(fy, 0, H - 1)
                    y1 = jnp.clip(fy + 1, 0, H - 1)
                    x0f = x0.astype(jnp.float32)
                    x1f = x1.astype(jnp.float32)
                    y0f = y0.astype(jnp.float32)
                    y1f = y1.astype(jnp.float32)
                    sl = pl.ds(m * 16, 16)
                    wa_v[sl] = (x1f - X) * (y1f - Y)
                    wb_v[sl] = (x1f - X) * (Y - y0f)
                    wc_v[sl] = (X - x0f) * (y1f - Y)
                    wd_v[sl] = (X - x0f) * (Y - y0f)
                    mk_v[sl] = jnp.where(inr, jnp.float32(1.0),
                                         jnp.float32(0.0))
                    ra = base_b + y0 * W
                    rb = base_b + y1 * W
                    idx_v[0, sl] = ra + x0
                    idx_v[1, sl] = rb + x0
                    idx_v[2, sl] = ra + x1
                    idx_v[3, sl] = rb + x1

                cps = []
                for m in range(NCHUNK):
                    sl = pl.ds(m * 16, 16)
                    any_in = jnp.max(mk_v[sl]) > 0.0

                    @pl.when(any_in)
                    def _fire(m=m, sl=sl):
                        for g, dst in enumerate((ga_v, gb_v, gc_v, gd_v)):
                            pltpu.async_copy(
                                im_hbm.at[idx_v.at[g, sl]],
                                dst.at[sl], sem)

                    cps.append(any_in)

                for m in range(NCHUNK):
                    sl = pl.ds(m * 16, 16)
                    any_in = cps[m]

                    @pl.when(any_in)
                    def _drain(m=m, sl=sl):
                        for dst in (ga_v, gb_v, gc_v, gd_v):
                            pltpu.make_async_copy(
                                im_hbm.at[idx_v.at[0, sl]],
                                dst.at[sl], sem).wait()

                    def blend(i, _, m=m):
                        iv = jnp.full((16,), i, jnp.int32)
                        wav = plsc.load_gather(wa_v, [iv])
                        wbv = plsc.load_gather(wb_v, [iv])
                        wcv = plsc.load_gather(wc_v, [iv])
                        wdv = plsc.load_gather(wd_v, [iv])
                        mv = plsc.load_gather(mk_v, [iv])
                        keep = mv > 0.5
                        for n in range(C // 16):
                            csl = pl.ds(n * 16, 16)
                            va = ga_v[i, csl]
                            vb = gb_v[i, csl]
                            vc = gc_v[i, csl]
                            vd = gd_v[i, csl]
                            val = va * wav + vb * wbv + vc * wcv + vd * wdv
                            out_v[pl.ds(i * C + n * 16, 16)] = jnp.where(
                                keep, val, jnp.float32(0.0))
                        return 0

                    lax.fori_loop(m * 16, m * 16 + 16, blend, 0)

                pix0 = base_b + r * W + h * HALF
                off = pl.multiple_of(pix0 * C, 8)
                pltpu.sync_copy(out_v, out_hbm.at[pl.ds(off, HALF * C)])
            return 0

        lax.fori_loop(0, ROWS_PER_W, do_row, 0)
        return 0

    lax.fori_loop(0, MB, do_image, 0)


@jax.jit
def _run(im2, th_flat):
    mesh = plsc.VectorSubcoreMesh(core_axis_name="c", subcore_axis_name="s")
    f = functools.partial(
        pl.kernel,
        mesh=mesh,
        compiler_params=pltpu.CompilerParams(
            needs_layout_passes=False, use_tc_tiling_on_sc=False),
        out_type=jax.ShapeDtypeStruct((NPIX * C,), jnp.float32),
        scratch_types=[
            pltpu.VMEM((MB * 6,), jnp.float32),
            pltpu.VMEM((4, HALF), jnp.int32),
            pltpu.VMEM((HALF,), jnp.float32),
            pltpu.VMEM((HALF,), jnp.float32),
            pltpu.VMEM((HALF,), jnp.float32),
            pltpu.VMEM((HALF,), jnp.float32),
            pltpu.VMEM((HALF,), jnp.float32),
            pltpu.VMEM((HALF, C), jnp.float32),
            pltpu.VMEM((HALF, C), jnp.float32),
            pltpu.VMEM((HALF, C), jnp.float32),
            pltpu.VMEM((HALF, C), jnp.float32),
            pltpu.VMEM((HALF * C,), jnp.float32),
            pltpu.SemaphoreType.DMA,
        ],
    )(_affine_kernel)
    return f(im2, th_flat)


def kernel(im, mb_size, thetas):
    im2 = im.reshape(NPIX, C)
    th_flat = thetas.reshape(MB * 6)
    flat = _run(im2, th_flat)
    return flat.reshape(MB, H, W, C)


# speedup vs baseline: 4.3518x; 1.0205x over previous
import functools

import jax
import jax.numpy as jnp
from jax import lax
from jax.experimental import pallas as pl
from jax.experimental.pallas import tpu as pltpu
from jax.experimental.pallas import tpu_sc as plsc

H = 224
W = 224
C = 96
MB = 8
NPIX = MB * H * W

NC = 2
NS = 16
NW = NC * NS
ROWS_PER_W = H // NW
HALF = W // 2
NCHUNK = HALF // 16

_STEP = 2.0 / 223.0


def _bf16r(x):
    u = lax.bitcast_convert_type(x, jnp.int32)
    rnd = lax.bitwise_and(lax.shift_right_logical(u, jnp.int32(16)),
                          jnp.int32(1))
    u = u + jnp.int32(0x7FFF) + rnd
    u = lax.bitwise_and(u, jnp.int32(-65536))
    return lax.bitcast_convert_type(u, jnp.float32)


def _affine_kernel(im_hbm, th_hbm, out_hbm, th_v, idx_v, wa_v, wb_v, wc_v,
                   wd_v, mk_v, ga_v, gb_v, gc_v, gd_v, out0_v, out1_v, sem,
                   wsem):
    wid = lax.axis_index("s") * NC + lax.axis_index("c")
    pltpu.sync_copy(th_hbm, th_v)
    obuf = (out0_v, out1_v)

    lanes = lax.iota(jnp.int32, 16)

    def do_image(b, _):
        tsel = [_bf16r(
            plsc.load_gather(th_v, [jnp.full((16,), b * 6 + k, jnp.int32)]))
                for k in range(6)]
        t0, t1, t2, t3, t4, t5 = tsel
        base_b = b * (H * W)

        def do_row(j, _):
            r = wid * ROWS_PER_W + j
            ygv = _bf16r(jnp.float32(-1.0) + jnp.full((16,), r, jnp.int32)
                         .astype(jnp.float32) * jnp.float32(_STEP))

            for h in range(2):
                out_v = obuf[h]
                for m in range(NCHUNK):
                    cols = lanes + (h * HALF + m * 16)
                    xgv = _bf16r(jnp.float32(-1.0)
                                 + cols.astype(jnp.float32)
                                 * jnp.float32(_STEP))
                    Xn = (t0 * xgv + t1 * ygv) + t2
                    Yn = (t3 * xgv + t4 * ygv) + t5
                    X = (Xn + 1.0) / 2.0 * jnp.float32(W)
                    Y = (Yn + 1.0) / 2.0 * jnp.float32(H)
                    fx = X.astype(jnp.int32)
                    fx = jnp.where(fx.astype(jnp.float32) > X, fx - 1, fx)
                    fy = Y.astype(jnp.int32)
                    fy = jnp.where(fy.astype(jnp.float32) > Y, fy - 1, fy)
                    inr = ((fx >= 0) & (fx <= W - 2)
                           & (fy >= 0) & (fy <= H - 2))
                    x0 = jnp.clip(fx, 0, W - 1)
                    x1 = jnp.clip(fx + 1, 0, W - 1)
                    y0 = jnp.clip(fy, 0, H - 1)
                    y1 = jnp.clip(fy + 1, 0, H - 1)
                    x0f = x0.astype(jnp.float32)
                    x1f = x1.astype(jnp.float32)
                    y0f = y0.astype(jnp.float32)
                    y1f = y1.astype(jnp.float32)
                    sl = pl.ds(m * 16, 16)
                    wa_v[sl] = (x1f - X) * (y1f - Y)
                    wb_v[sl] = (x1f - X) * (Y - y0f)
                    wc_v[sl] = (X - x0f) * (y1f - Y)
                    wd_v[sl] = (X - x0f) * (Y - y0f)
                    mk_v[sl] = jnp.where(inr, jnp.float32(1.0),
                                         jnp.float32(0.0))
                    ra = base_b + y0 * W
                    rb = base_b + y1 * W
                    idx_v[0, sl] = ra + x0
                    idx_v[1, sl] = rb + x0
                    idx_v[2, sl] = ra + x1
                    idx_v[3, sl] = rb + x1
                    any_in = jnp.max(jnp.where(inr, 1, 0)) > 0

                    @pl.when(any_in)
                    def _fire(sl=sl):
                        for g, dst in enumerate((ga_v, gb_v, gc_v, gd_v)):
                            pltpu.async_copy(
                                im_hbm.at[idx_v.at[g, sl]],
                                dst.at[sl], sem)

                gidx = (b * ROWS_PER_W + j) * 2 + h

                @pl.when(gidx >= 2)
                def _wb_drain():
                    pltpu.make_async_copy(
                        out_v, out_hbm.at[pl.ds(0, HALF * C)], wsem).wait()

                for m in range(NCHUNK):
                    sl = pl.ds(m * 16, 16)
                    any_in = jnp.max(mk_v[sl]) > 0.0

                    @pl.when(any_in)
                    def _drain(m=m, sl=sl):
                        for dst in (ga_v, gb_v, gc_v, gd_v):
                            pltpu.make_async_copy(
                                im_hbm.at[idx_v.at[0, sl]],
                                dst.at[sl], sem).wait()

                    def blend(i, _, m=m):
                        iv = jnp.full((16,), i, jnp.int32)
                        wav = plsc.load_gather(wa_v, [iv])
                        wbv = plsc.load_gather(wb_v, [iv])
                        wcv = plsc.load_gather(wc_v, [iv])
                        wdv = plsc.load_gather(wd_v, [iv])
                        mv = plsc.load_gather(mk_v, [iv])
                        keep = mv > 0.5
                        for n in range(C // 16):
                            csl = pl.ds(n * 16, 16)
                            va = ga_v[i, csl]
                            vb = gb_v[i, csl]
                            vc = gc_v[i, csl]
                            vd = gd_v[i, csl]
                            val = va * wav + vb * wbv + vc * wcv + vd * wdv
                            out_v[pl.ds(i * C + n * 16, 16)] = jnp.where(
                                keep, val, jnp.float32(0.0))
                        return 0

                    lax.fori_loop(m * 16, m * 16 + 16, blend, 0)

                pix0 = base_b + r * W + h * HALF
                off = pl.multiple_of(pix0 * C, 8)
                pltpu.async_copy(out_v, out_hbm.at[pl.ds(off, HALF * C)],
                                 wsem)
            return 0

        lax.fori_loop(0, ROWS_PER_W, do_row, 0)
        return 0

    lax.fori_loop(0, MB, do_image, 0)
    for ob in obuf:
        pltpu.make_async_copy(ob, out_hbm.at[pl.ds(0, HALF * C)],
                              wsem).wait()


@jax.jit
def _run(im2, th_flat):
    mesh = plsc.VectorSubcoreMesh(core_axis_name="c", subcore_axis_name="s")
    f = functools.partial(
        pl.kernel,
        mesh=mesh,
        compiler_params=pltpu.CompilerParams(
            needs_layout_passes=False, use_tc_tiling_on_sc=False),
        out_type=jax.ShapeDtypeStruct((NPIX * C,), jnp.float32),
        scratch_types=[
            pltpu.VMEM((MB * 6,), jnp.float32),
            pltpu.VMEM((4, HALF), jnp.int32),
            pltpu.VMEM((HALF,), jnp.float32),
            pltpu.VMEM((HALF,), jnp.float32),
            pltpu.VMEM((HALF,), jnp.float32),
            pltpu.VMEM((HALF,), jnp.float32),
            pltpu.VMEM((HALF,), jnp.float32),
            pltpu.VMEM((HALF, C), jnp.float32),
            pltpu.VMEM((HALF, C), jnp.float32),
            pltpu.VMEM((HALF, C), jnp.float32),
            pltpu.VMEM((HALF, C), jnp.float32),
            pltpu.VMEM((HALF * C,), jnp.float32),
            pltpu.VMEM((HALF * C,), jnp.float32),
            pltpu.SemaphoreType.DMA,
            pltpu.SemaphoreType.DMA,
        ],
    )(_affine_kernel)
    return f(im2, th_flat)


def kernel(im, mb_size, thetas):
    im2 = im.reshape(NPIX, C)
    th_flat = thetas.reshape(MB * 6)
    flat = _run(im2, th_flat)
    return flat.reshape(MB, H, W, C)
